# Initial kernel scaffold; baseline (speedup 1.0000x reference)
import jax
import jax.numpy as jnp
from jax.experimental import pallas as pl


def kernel(boxes, scores):
    raise NotImplementedError("write your pallas kernel here")


# speedup vs baseline: 29.8092x; 29.8092x over previous
import functools

import jax
import jax.numpy as jnp
from jax.experimental import pallas as pl
from jax.experimental.pallas import tpu as pltpu

_N = 20000
_MAX_DET = 100
_IOU_THRESH = 0.5
_SCORE_THRESH = 0.05

_ROWS = 160
_NPAD = _ROWS * 128
_BIG = 2**30


def _nms_body(x1_ref, y1_ref, x2_ref, y2_ref, s_ref, out_ref):
    x1 = x1_ref[...]
    y1 = y1_ref[...]
    x2 = x2_ref[...]
    y2 = y2_ref[...]
    area = (x2 - x1) * (y2 - y1)
    s0 = jnp.where(s_ref[...] > _SCORE_THRESH, s_ref[...], -jnp.inf)

    gidx = jax.lax.broadcasted_iota(jnp.int32, (_ROWS, 128), 0) * 128 + \
        jax.lax.broadcasted_iota(jnp.int32, (_ROWS, 128), 1)
    lane = jax.lax.broadcasted_iota(jnp.int32, (1, 128), 1)

    def step(i, s):
        m = jnp.max(s)
        sel_idx = jnp.min(jnp.where(s == m, gidx, _BIG))
        onehot = (gidx == sel_idx).astype(jnp.float32)
        bx1 = jnp.sum(onehot * x1)
        by1 = jnp.sum(onehot * y1)
        bx2 = jnp.sum(onehot * x2)
        by2 = jnp.sum(onehot * y2)
        barea = (bx2 - bx1) * (by2 - by1)

        ix1 = jnp.maximum(bx1, x1)
        iy1 = jnp.maximum(by1, y1)
        ix2 = jnp.minimum(bx2, x2)
        iy2 = jnp.minimum(by2, y2)
        inter = jnp.maximum(ix2 - ix1, 0.0) * jnp.maximum(iy2 - iy1, 0.0)
        union = barea + area - inter
        iou = inter / jnp.maximum(union, 1e-9)
        s_new = jnp.where(iou > _IOU_THRESH, -jnp.inf, s)

        valid = m > -jnp.inf
        vals = jnp.where(lane == 0, bx1, 0.0)
        vals = jnp.where(lane == 1, by1, vals)
        vals = jnp.where(lane == 2, bx2, vals)
        vals = jnp.where(lane == 3, by2, vals)
        vals = jnp.where(lane == 4, m, vals)
        vals = jnp.where(valid, vals, 0.0)
        out_ref[pl.ds(i, 1), :] = vals
        return s_new

    jax.lax.fori_loop(0, _MAX_DET, step, s0)


@jax.jit
def kernel(boxes, scores):
    x1 = jnp.zeros((_NPAD,), jnp.float32).at[:_N].set(boxes[:, 0])
    y1 = jnp.zeros((_NPAD,), jnp.float32).at[:_N].set(boxes[:, 1])
    x2 = jnp.zeros((_NPAD,), jnp.float32).at[:_N].set(boxes[:, 2])
    y2 = jnp.zeros((_NPAD,), jnp.float32).at[:_N].set(boxes[:, 3])
    s = jnp.zeros((_NPAD,), jnp.float32).at[:_N].set(scores)
    shape2d = (_ROWS, 128)
    args = [a.reshape(shape2d) for a in (x1, y1, x2, y2, s)]

    out = pl.pallas_call(
        _nms_body,
        out_shape=jax.ShapeDtypeStruct((_MAX_DET, 128), jnp.float32),
    )(*args)
    return out[:, :5]
